# baseline (speedup 1.0000x reference)
import functools

import jax
import jax.numpy as jnp
from jax import lax
from jax.experimental import pallas as pl
from jax.experimental.pallas import tpu as pltpu
from jax.experimental.pallas import tpu_sc as plsc

TOPK_K = 8
B, LX, LM, D = 16, 256, 4096, 512
NC, NS, L = 2, 16, 16


NJ = 2
LMB = LM // NJ


def _scores_body(x_ref, mem_ref, s_ref):
    x = x_ref[0]
    mem = mem_ref[0]
    dot_t = lax.dot_general(
        x, mem, (((1,), (1,)), ((), ())),
        preferred_element_type=jnp.float32,
    )
    inv_na = 1.0 / jnp.sqrt(jnp.sum(x * x, axis=1, keepdims=True))
    best = jnp.max(dot_t * inv_na, axis=0, keepdims=True)
    ones = jnp.ones((1, D), jnp.float32)
    nb2 = lax.dot_general(
        ones, mem * mem, (((1,), (1,)), ((), ())),
        preferred_element_type=jnp.float32,
        precision=lax.Precision.HIGHEST,
    )
    s_ref[0] = best * jnp.abs(best) / nb2


def _tc_scores(embedded_x, embedded_memory):
    scores3 = pl.pallas_call(
        _scores_body,
        grid=(B, NJ),
        in_specs=[
            pl.BlockSpec((1, LX, D), lambda b, j: (b, 0, 0)),
            pl.BlockSpec((1, LMB, D), lambda b, j: (b, j, 0)),
        ],
        out_specs=pl.BlockSpec((1, 1, LMB), lambda b, j: (b, 0, j)),
        out_shape=jax.ShapeDtypeStruct((B, 1, LM), jnp.float32),
    )(embedded_x, embedded_memory)
    return scores3.reshape(B, LM)


def _sc_topk_body(scores_hbm, mem_hbm, q_hbm, i_hbm,
                  s_v, fidx_v, rows_v, iout_v, sem):
    wid = lax.axis_index("s") * NC + lax.axis_index("c")

    @pl.when(wid < B)
    def _():
        b = wid
        pltpu.sync_copy(scores_hbm.at[b], s_v)
        iota = lax.iota(jnp.int32, L)

        def step(i, carry):
            bv, bi = carry
            chunk = s_v[pl.ds(i * L, L)]
            cv, ci = plsc.sort_key_val(chunk, iota + i * L, descending=True)
            take = (cv > bv) | ((cv == bv) & (ci < bi))
            nv = jnp.where(take, cv, bv)
            ni = jnp.where(take, ci, bi)
            nv, ni = plsc.sort_key_val(nv, ni)
            return nv, ni

        init = (jnp.full((L,), -jnp.inf, jnp.float32),
                jnp.zeros((L,), jnp.int32))
        bv, bi = lax.fori_loop(0, LM // L, step, init)
        ri = lax.rev(bi, (0,))
        iout_v[...] = ri
        fidx_v[...] = ri + b * LM
        pltpu.async_copy(mem_hbm.at[fidx_v], rows_v, sem).wait()
        pltpu.sync_copy(rows_v.at[pl.ds(0, TOPK_K)], q_hbm.at[b])
        pltpu.sync_copy(iout_v.at[pl.ds(0, TOPK_K)],
                        i_hbm.at[pl.ds(b * TOPK_K, TOPK_K)])


_sc_topk = functools.partial(
    pl.kernel,
    out_type=[
        jax.ShapeDtypeStruct((B, TOPK_K, D), jnp.float32),
        jax.ShapeDtypeStruct((B * TOPK_K,), jnp.int32),
    ],
    mesh=plsc.VectorSubcoreMesh(
        core_axis_name="c", subcore_axis_name="s",
        num_cores=NC, num_subcores=NS,
    ),
    compiler_params=pltpu.CompilerParams(needs_layout_passes=False),
    scratch_types=[
        pltpu.VMEM((LM,), jnp.float32),
        pltpu.VMEM((L,), jnp.int32),
        pltpu.VMEM((L, D), jnp.float32),
        pltpu.VMEM((L,), jnp.int32),
        pltpu.SemaphoreType.DMA,
    ],
)(_sc_topk_body)


def kernel(embedded_x, embedded_memory):
    scores = _tc_scores(embedded_x, embedded_memory)
    mem_flat = embedded_memory.reshape(B * LM, D)
    querys, ret_idx = _sc_topk(scores, mem_flat)
    return tuple(querys[:, i, :] for i in range(TOPK_K)) + (
        ret_idx.reshape(B, TOPK_K),)


# speedup vs baseline: 1.0251x; 1.0251x over previous
import functools

import jax
import jax.numpy as jnp
from jax import lax
from jax.experimental import pallas as pl
from jax.experimental.pallas import tpu as pltpu
from jax.experimental.pallas import tpu_sc as plsc

TOPK_K = 8
B, LX, LM, D = 16, 256, 4096, 512
NC, NS, L = 2, 16, 16


NJ = 1
LMB = LM // NJ


def _scores_body(x_ref, mem_ref, s_ref):
    x = x_ref[0]
    mem = mem_ref[0]
    dot_t = lax.dot_general(
        x, mem, (((1,), (1,)), ((), ())),
        preferred_element_type=jnp.float32,
    )
    inv_na = 1.0 / jnp.sqrt(jnp.sum(x * x, axis=1, keepdims=True))
    best = jnp.max(dot_t * inv_na, axis=0, keepdims=True)
    ones = jnp.ones((1, D), jnp.float32)
    nb2 = lax.dot_general(
        ones, mem * mem, (((1,), (1,)), ((), ())),
        preferred_element_type=jnp.float32,
        precision=lax.Precision.HIGHEST,
    )
    s_ref[0] = best * jnp.abs(best) / nb2


def _tc_scores(embedded_x, embedded_memory):
    scores3 = pl.pallas_call(
        _scores_body,
        grid=(B, NJ),
        in_specs=[
            pl.BlockSpec((1, LX, D), lambda b, j: (b, 0, 0)),
            pl.BlockSpec((1, LMB, D), lambda b, j: (b, j, 0)),
        ],
        out_specs=pl.BlockSpec((1, 1, LMB), lambda b, j: (b, 0, j)),
        out_shape=jax.ShapeDtypeStruct((B, 1, LM), jnp.float32),
    )(embedded_x, embedded_memory)
    return scores3.reshape(B, LM)


def _sc_topk_body(scores_hbm, mem_hbm, q_hbm, i_hbm,
                  s_v, fidx_v, rows_v, iout_v, sem):
    wid = lax.axis_index("s") * NC + lax.axis_index("c")

    @pl.when(wid < B)
    def _():
        b = wid
        pltpu.sync_copy(scores_hbm.at[b], s_v)
        iota = lax.iota(jnp.int32, L)

        def step(i, carry):
            bv, bi = carry
            chunk = s_v[pl.ds(i * L, L)]
            cv, ci = plsc.sort_key_val(chunk, iota + i * L, descending=True)
            take = (cv > bv) | ((cv == bv) & (ci < bi))
            nv = jnp.where(take, cv, bv)
            ni = jnp.where(take, ci, bi)
            nv, ni = plsc.sort_key_val(nv, ni)
            return nv, ni

        init = (jnp.full((L,), -jnp.inf, jnp.float32),
                jnp.zeros((L,), jnp.int32))
        bv, bi = lax.fori_loop(0, LM // L, step, init)
        ri = lax.rev(bi, (0,))
        iout_v[...] = ri
        fidx_v[...] = ri + b * LM
        pltpu.async_copy(mem_hbm.at[fidx_v], rows_v, sem).wait()
        pltpu.sync_copy(rows_v.at[pl.ds(0, TOPK_K)], q_hbm.at[b])
        pltpu.sync_copy(iout_v.at[pl.ds(0, TOPK_K)],
                        i_hbm.at[pl.ds(b * TOPK_K, TOPK_K)])


_sc_topk = functools.partial(
    pl.kernel,
    out_type=[
        jax.ShapeDtypeStruct((B, TOPK_K, D), jnp.float32),
        jax.ShapeDtypeStruct((B * TOPK_K,), jnp.int32),
    ],
    mesh=plsc.VectorSubcoreMesh(
        core_axis_name="c", subcore_axis_name="s",
        num_cores=NC, num_subcores=NS,
    ),
    compiler_params=pltpu.CompilerParams(needs_layout_passes=False),
    scratch_types=[
        pltpu.VMEM((LM,), jnp.float32),
        pltpu.VMEM((L,), jnp.int32),
        pltpu.VMEM((L, D), jnp.float32),
        pltpu.VMEM((L,), jnp.int32),
        pltpu.SemaphoreType.DMA,
    ],
)(_sc_topk_body)


def kernel(embedded_x, embedded_memory):
    scores = _tc_scores(embedded_x, embedded_memory)
    mem_flat = embedded_memory.reshape(B * LM, D)
    querys, ret_idx = _sc_topk(scores, mem_flat)
    return tuple(querys[:, i, :] for i in range(TOPK_K)) + (
        ret_idx.reshape(B, TOPK_K),)


# speedup vs baseline: 2.1119x; 2.0603x over previous
import functools

import jax
import jax.numpy as jnp
from jax import lax
from jax.experimental import pallas as pl
from jax.experimental.pallas import tpu as pltpu
from jax.experimental.pallas import tpu_sc as plsc

TOPK_K = 8
B, LX, LM, D = 16, 256, 4096, 512
NC, NS, L = 2, 16, 16


NJ = 1
LMB = LM // NJ


def _scores_body(x_ref, mem_ref, s_ref):
    x = x_ref[0]
    mem = mem_ref[0]
    nb2 = jnp.sum(mem * mem, axis=1, keepdims=True).reshape(1, LMB)
    dot_t = lax.dot_general(
        x, mem, (((1,), (1,)), ((), ())),
        preferred_element_type=jnp.float32,
    )
    inv_na = 1.0 / jnp.sqrt(jnp.sum(x * x, axis=1, keepdims=True))
    best = jnp.max(dot_t * inv_na, axis=0, keepdims=True)
    s_ref[0] = best * jnp.abs(best) / nb2


def _tc_scores(embedded_x, embedded_memory):
    scores3 = pl.pallas_call(
        _scores_body,
        grid=(B, NJ),
        in_specs=[
            pl.BlockSpec((1, LX, D), lambda b, j: (b, 0, 0)),
            pl.BlockSpec((1, LMB, D), lambda b, j: (b, j, 0)),
        ],
        out_specs=pl.BlockSpec((1, 1, LMB), lambda b, j: (b, 0, j)),
        out_shape=jax.ShapeDtypeStruct((B, 1, LM), jnp.float32),
    )(embedded_x, embedded_memory)
    return scores3.reshape(B, LM)


def _sc_topk_body(scores_hbm, mem_hbm, q_hbm, i_hbm,
                  s_v, fidx_v, rows_v, iout_v, sem):
    wid = lax.axis_index("s") * NC + lax.axis_index("c")

    @pl.when(wid < B)
    def _():
        b = wid
        pltpu.sync_copy(scores_hbm.at[b], s_v)
        iota = lax.iota(jnp.int32, L)

        def step(i, carry):
            bv, bi = carry
            chunk = s_v[pl.ds(i * L, L)]
            cv, ci = plsc.sort_key_val(chunk, iota + i * L, descending=True)
            take = (cv > bv) | ((cv == bv) & (ci < bi))
            nv = jnp.where(take, cv, bv)
            ni = jnp.where(take, ci, bi)
            nv, ni = plsc.sort_key_val(nv, ni)
            return nv, ni

        init = (jnp.full((L,), -jnp.inf, jnp.float32),
                jnp.zeros((L,), jnp.int32))
        bv, bi = lax.fori_loop(0, LM // L, step, init)
        ri = lax.rev(bi, (0,))
        iout_v[...] = ri
        fidx_v[...] = ri + b * LM
        pltpu.async_copy(mem_hbm.at[fidx_v], rows_v, sem).wait()
        pltpu.sync_copy(rows_v.at[pl.ds(0, TOPK_K)], q_hbm.at[b])
        pltpu.sync_copy(iout_v.at[pl.ds(0, TOPK_K)],
                        i_hbm.at[pl.ds(b * TOPK_K, TOPK_K)])


_sc_topk = functools.partial(
    pl.kernel,
    out_type=[
        jax.ShapeDtypeStruct((B, TOPK_K, D), jnp.float32),
        jax.ShapeDtypeStruct((B * TOPK_K,), jnp.int32),
    ],
    mesh=plsc.VectorSubcoreMesh(
        core_axis_name="c", subcore_axis_name="s",
        num_cores=NC, num_subcores=NS,
    ),
    compiler_params=pltpu.CompilerParams(needs_layout_passes=False),
    scratch_types=[
        pltpu.VMEM((LM,), jnp.float32),
        pltpu.VMEM((L,), jnp.int32),
        pltpu.VMEM((L, D), jnp.float32),
        pltpu.VMEM((L,), jnp.int32),
        pltpu.SemaphoreType.DMA,
    ],
)(_sc_topk_body)


def kernel(embedded_x, embedded_memory):
    scores = _tc_scores(embedded_x, embedded_memory)
    mem_flat = embedded_memory.reshape(B * LM, D)
    querys, ret_idx = _sc_topk(scores, mem_flat)
    return tuple(querys[:, i, :] for i in range(TOPK_K)) + (
        ret_idx.reshape(B, TOPK_K),)


# speedup vs baseline: 2.1153x; 1.0016x over previous
import functools

import jax
import jax.numpy as jnp
from jax import lax
from jax.experimental import pallas as pl
from jax.experimental.pallas import tpu as pltpu
from jax.experimental.pallas import tpu_sc as plsc

TOPK_K = 8
B, LX, LM, D = 16, 256, 4096, 512
NC, NS, L = 2, 16, 16


NJ = 1
LMB = LM // NJ


def _scores_body(x_ref, mem_ref, s_ref):
    x = x_ref[0]
    mem = mem_ref[0]
    nb2 = jnp.sum(mem * mem, axis=1, keepdims=True).reshape(1, LMB)
    dot_t = lax.dot_general(
        x, mem, (((1,), (1,)), ((), ())),
        preferred_element_type=jnp.float32,
    )
    inv_na = 1.0 / jnp.sqrt(jnp.sum(x * x, axis=1, keepdims=True))
    best = jnp.max(dot_t * inv_na, axis=0, keepdims=True)
    s_ref[0] = best * jnp.abs(best) / nb2


def _tc_scores(embedded_x, embedded_memory):
    scores3 = pl.pallas_call(
        _scores_body,
        grid=(B, NJ),
        in_specs=[
            pl.BlockSpec((1, LX, D), lambda b, j: (b, 0, 0)),
            pl.BlockSpec((1, LMB, D), lambda b, j: (b, j, 0)),
        ],
        out_specs=pl.BlockSpec((1, 1, LMB), lambda b, j: (b, 0, j)),
        out_shape=jax.ShapeDtypeStruct((B, 1, LM), jnp.float32),
    )(embedded_x, embedded_memory)
    return scores3.reshape(B, LM)


def _sc_topk_body(scores_hbm, mem_hbm, q_hbm, i_hbm,
                  s_v, fidx_v, rows_v, iout_v, sem):
    wid = lax.axis_index("s") * NC + lax.axis_index("c")

    @pl.when(wid < B)
    def _():
        b = wid
        pltpu.sync_copy(scores_hbm.at[b], s_v)
        iota = lax.iota(jnp.int32, L)

        def step(i, carry):
            bv, bi = carry
            chunk = s_v[pl.ds(i * L, L)]
            cv, ci = plsc.sort_key_val(chunk, iota + i * L, descending=True)
            take = (cv > bv) | ((cv == bv) & (ci < bi))
            nv = jnp.where(take, cv, bv)
            ni = jnp.where(take, ci, bi)
            nv, ni = plsc.sort_key_val(nv, ni)
            return nv, ni

        init = (jnp.full((L,), -jnp.inf, jnp.float32),
                jnp.zeros((L,), jnp.int32))
        bv, bi = lax.fori_loop(0, LM // L, step, init)
        ri = lax.rev(bi, (0,))
        iout_v[...] = ri
        fidx_v[...] = ri + b * LM
        pltpu.async_copy(mem_hbm.at[fidx_v.at[pl.ds(0, TOPK_K)]],
                         rows_v.at[pl.ds(0, TOPK_K)], sem).wait()
        pltpu.sync_copy(rows_v.at[pl.ds(0, TOPK_K)], q_hbm.at[b])
        pltpu.sync_copy(iout_v.at[pl.ds(0, TOPK_K)],
                        i_hbm.at[pl.ds(b * TOPK_K, TOPK_K)])


_sc_topk = functools.partial(
    pl.kernel,
    out_type=[
        jax.ShapeDtypeStruct((B, TOPK_K, D), jnp.float32),
        jax.ShapeDtypeStruct((B * TOPK_K,), jnp.int32),
    ],
    mesh=plsc.VectorSubcoreMesh(
        core_axis_name="c", subcore_axis_name="s",
        num_cores=NC, num_subcores=NS,
    ),
    compiler_params=pltpu.CompilerParams(needs_layout_passes=False),
    scratch_types=[
        pltpu.VMEM((LM,), jnp.float32),
        pltpu.VMEM((L,), jnp.int32),
        pltpu.VMEM((L, D), jnp.float32),
        pltpu.VMEM((L,), jnp.int32),
        pltpu.SemaphoreType.DMA,
    ],
)(_sc_topk_body)


def kernel(embedded_x, embedded_memory):
    scores = _tc_scores(embedded_x, embedded_memory)
    mem_flat = embedded_memory.reshape(B * LM, D)
    querys, ret_idx = _sc_topk(scores, mem_flat)
    return tuple(querys[:, i, :] for i in range(TOPK_K)) + (
        ret_idx.reshape(B, TOPK_K),)


# speedup vs baseline: 2.2508x; 1.0641x over previous
import functools

import jax
import jax.numpy as jnp
from jax import lax
from jax.experimental import pallas as pl
from jax.experimental.pallas import tpu as pltpu
from jax.experimental.pallas import tpu_sc as plsc

TOPK_K = 8
B, LX, LM, D = 16, 256, 4096, 512
NC, NS, L = 2, 16, 16


NJ = 1
LMB = LM // NJ


def _scores_body(x_ref, mem_ref, s_ref):
    x = x_ref[0]
    mem = mem_ref[0]
    nb2 = jnp.sum(mem * mem, axis=1, keepdims=True).reshape(1, LMB)
    dot_t = lax.dot_general(
        x, mem, (((1,), (1,)), ((), ())),
        preferred_element_type=jnp.float32,
    )
    inv_na = 1.0 / jnp.sqrt(jnp.sum(x * x, axis=1, keepdims=True))
    best = jnp.max(dot_t * inv_na, axis=0, keepdims=True)
    s_ref[0] = best * jnp.abs(best) / nb2


def _tc_scores(embedded_x, embedded_memory):
    scores3 = pl.pallas_call(
        _scores_body,
        grid=(B, NJ),
        in_specs=[
            pl.BlockSpec((1, LX, D), lambda b, j: (b, 0, 0)),
            pl.BlockSpec((1, LMB, D), lambda b, j: (b, j, 0)),
        ],
        out_specs=pl.BlockSpec((1, 1, LMB), lambda b, j: (b, 0, j)),
        out_shape=jax.ShapeDtypeStruct((B, 1, LM), jnp.float32),
    )(embedded_x, embedded_memory)
    return scores3.reshape(B, LM)


def _sc_topk_body(scores_hbm, mem_hbm, *rest):
    qs = rest[:TOPK_K]
    i_hbm = rest[TOPK_K]
    s_v, fidx_v, rows_v, iout_v, sem = rest[TOPK_K + 1:]
    wid = lax.axis_index("s") * NC + lax.axis_index("c")

    @pl.when(wid < B)
    def _():
        b = wid
        pltpu.sync_copy(scores_hbm.at[b], s_v)
        iota = lax.iota(jnp.int32, L)

        def step(i, carry):
            bv, bi = carry
            chunk = s_v[pl.ds(i * L, L)]
            cv, ci = plsc.sort_key_val(chunk, iota + i * L, descending=True)
            take = (cv > bv) | ((cv == bv) & (ci < bi))
            nv = jnp.where(take, cv, bv)
            ni = jnp.where(take, ci, bi)
            nv, ni = plsc.sort_key_val(nv, ni)
            return nv, ni

        init = (jnp.full((L,), -jnp.inf, jnp.float32),
                jnp.zeros((L,), jnp.int32))
        bv, bi = lax.fori_loop(0, LM // L, step, init)
        ri = lax.rev(bi, (0,))
        iout_v[...] = ri
        fidx_v[...] = ri + b * LM
        pltpu.async_copy(mem_hbm.at[fidx_v.at[pl.ds(0, TOPK_K)]],
                         rows_v.at[pl.ds(0, TOPK_K)], sem).wait()
        copies = [
            pltpu.async_copy(rows_v.at[pl.ds(i, 1)], q.at[pl.ds(b, 1)], sem)
            for i, q in enumerate(qs)
        ]
        for cp in copies:
            cp.wait()
        pltpu.sync_copy(iout_v.at[pl.ds(0, TOPK_K)],
                        i_hbm.at[pl.ds(b * TOPK_K, TOPK_K)])


_sc_topk = functools.partial(
    pl.kernel,
    out_type=[jax.ShapeDtypeStruct((B, D), jnp.float32)] * TOPK_K
    + [jax.ShapeDtypeStruct((B * TOPK_K,), jnp.int32)],
    mesh=plsc.VectorSubcoreMesh(
        core_axis_name="c", subcore_axis_name="s",
        num_cores=NC, num_subcores=NS,
    ),
    compiler_params=pltpu.CompilerParams(needs_layout_passes=False),
    scratch_types=[
        pltpu.VMEM((LM,), jnp.float32),
        pltpu.VMEM((L,), jnp.int32),
        pltpu.VMEM((L, D), jnp.float32),
        pltpu.VMEM((L,), jnp.int32),
        pltpu.SemaphoreType.DMA,
    ],
)(_sc_topk_body)


def kernel(embedded_x, embedded_memory):
    scores = _tc_scores(embedded_x, embedded_memory)
    mem_flat = embedded_memory.reshape(B * LM, D)
    outs = _sc_topk(scores, mem_flat)
    return tuple(outs[:TOPK_K]) + (outs[TOPK_K].reshape(B, TOPK_K),)


# speedup vs baseline: 2.3153x; 1.0287x over previous
import functools

import jax
import jax.numpy as jnp
from jax import lax
from jax.experimental import pallas as pl
from jax.experimental.pallas import tpu as pltpu
from jax.experimental.pallas import tpu_sc as plsc

TOPK_K = 8
B, LX, LM, D = 16, 256, 4096, 512
NC, NS, L = 2, 16, 16


NJ = 1
LMB = LM // NJ


def _scores_body(x_ref, mem_ref, s_ref):
    x = x_ref[0]
    mem = mem_ref[0]
    nb2 = jnp.sum(mem * mem, axis=1, keepdims=True).reshape(1, LMB)
    dot_t = lax.dot_general(
        x, mem, (((1,), (1,)), ((), ())),
        preferred_element_type=jnp.float32,
    )
    inv_na = 1.0 / jnp.sqrt(jnp.sum(x * x, axis=1, keepdims=True))
    best = jnp.max(dot_t * inv_na, axis=0, keepdims=True)
    s_ref[0] = best * jnp.abs(best) / nb2


def _tc_scores(embedded_x, embedded_memory):
    scores3 = pl.pallas_call(
        _scores_body,
        grid=(B, NJ),
        in_specs=[
            pl.BlockSpec((1, LX, D), lambda b, j: (b, 0, 0)),
            pl.BlockSpec((1, LMB, D), lambda b, j: (b, j, 0)),
        ],
        out_specs=pl.BlockSpec((1, 1, LMB), lambda b, j: (b, 0, j)),
        out_shape=jax.ShapeDtypeStruct((B, 1, LM), jnp.float32),
    )(embedded_x, embedded_memory)
    return scores3


def _sc_topk_body(scores_hbm, mem_hbm, *rest):
    qs = rest[:TOPK_K]
    i_hbm = rest[TOPK_K]
    s_v, fidx_v, rows_v, iout_v, sem = rest[TOPK_K + 1:]
    wid = lax.axis_index("s") * NC + lax.axis_index("c")

    @pl.when(wid < B)
    def _():
        b = wid
        pltpu.sync_copy(scores_hbm.at[b, 0], s_v)
        iota = lax.iota(jnp.int32, L)

        def step(i, carry):
            bv, bi = carry
            chunk = s_v[pl.ds(i * L, L)]
            cv, ci = plsc.sort_key_val(chunk, iota + i * L, descending=True)
            take = (cv > bv) | ((cv == bv) & (ci < bi))
            nv = jnp.where(take, cv, bv)
            ni = jnp.where(take, ci, bi)
            nv, ni = plsc.sort_key_val(nv, ni)
            return nv, ni

        init = (jnp.full((L,), -jnp.inf, jnp.float32),
                jnp.zeros((L,), jnp.int32))
        bv, bi = lax.fori_loop(0, LM // L, step, init)
        ri = lax.rev(bi, (0,))
        iout_v[...] = ri
        fidx_v[...] = ri + b * LM
        pltpu.async_copy(mem_hbm.at[fidx_v.at[pl.ds(0, TOPK_K)]],
                         rows_v.at[pl.ds(0, TOPK_K)], sem).wait()
        copies = [
            pltpu.async_copy(rows_v.at[pl.ds(i, 1)], q.at[pl.ds(b, 1)], sem)
            for i, q in enumerate(qs)
        ]
        for cp in copies:
            cp.wait()
        pltpu.sync_copy(iout_v.at[pl.ds(0, TOPK_K)],
                        i_hbm.at[pl.ds(b * TOPK_K, TOPK_K)])


_sc_topk = functools.partial(
    pl.kernel,
    out_type=[jax.ShapeDtypeStruct((B, D), jnp.float32)] * TOPK_K
    + [jax.ShapeDtypeStruct((B * TOPK_K,), jnp.int32)],
    mesh=plsc.VectorSubcoreMesh(
        core_axis_name="c", subcore_axis_name="s",
        num_cores=NC, num_subcores=NS,
    ),
    compiler_params=pltpu.CompilerParams(needs_layout_passes=False),
    scratch_types=[
        pltpu.VMEM((LM,), jnp.float32),
        pltpu.VMEM((L,), jnp.int32),
        pltpu.VMEM((L, D), jnp.float32),
        pltpu.VMEM((L,), jnp.int32),
        pltpu.SemaphoreType.DMA,
    ],
)(_sc_topk_body)


def kernel(embedded_x, embedded_memory):
    scores = _tc_scores(embedded_x, embedded_memory)
    mem_flat = embedded_memory.reshape(B * LM, D)
    outs = _sc_topk(scores, mem_flat)
    return tuple(outs[:TOPK_K]) + (outs[TOPK_K].reshape(B, TOPK_K),)
